# baseline (speedup 1.0000x reference)
import functools

import jax
import jax.numpy as jnp
from jax import lax
from jax.experimental import pallas as pl
from jax.experimental.pallas import tpu as pltpu
from jax.experimental.pallas import tpu_sc as plsc

_F32_EPS = float(jnp.finfo(jnp.float32).eps)
_BN_EPS = 1e-5


def _stage_a_body(x_ref, yt_ref, idx_ref, wexp_ref, *, s_total):
    b = pl.program_id(0)

    x = x_ref[0]
    yt = yt_ref[0]
    s = yt.shape[1]

    xx = jnp.sum(x * x, axis=1, keepdims=True)
    yy = 0.25 * jnp.sum(yt * yt, axis=0, keepdims=True)
    xy = jnp.dot(x, yt, preferred_element_type=jnp.float32)
    d = jnp.maximum((xx + yy) + xy, 0.0)

    colf = jax.lax.broadcasted_iota(jnp.int32, d.shape, 1).astype(jnp.float32)
    big = jnp.float32(s)
    ms, idxs = [], []
    for k in range(3):
        m = jnp.min(d, axis=1, keepdims=True)
        i = jnp.min(jnp.where(d == m, colf, big), axis=1, keepdims=True)
        ms.append(m)
        idxs.append(i)
        if k < 2:
            d = jnp.where(colf == i, jnp.inf, d)

    r0 = 1.0 / (jnp.sqrt(ms[0]) + _F32_EPS)
    r1 = 1.0 / (jnp.sqrt(ms[1]) + _F32_EPS)
    r2 = 1.0 / (jnp.sqrt(ms[2]) + _F32_EPS)
    norm = r0 + r1 + r2
    w0 = r0 / norm
    w1 = r1 / norm
    w2 = r2 / norm

    off = (b * s_total).astype(jnp.int32)
    ii = jnp.concatenate([idxs[0], idxs[1], idxs[2]],
                         axis=1).astype(jnp.int32) + off
    idx_ref[...] = ii

    ones16 = jnp.ones((1, 16), jnp.float32)
    wexp_ref[...] = jnp.concatenate(
        [w0 * ones16, w1 * ones16, w2 * ones16], axis=1)


def _stage_a2_body(p1_ref, it_ref, w0a_ref, w0b_ref, y0_ref, st_ref, acc_ref,
                   *, n_tiles):
    t = pl.program_id(0)
    y0 = (jnp.dot(p1_ref[...], w0a_ref[...], preferred_element_type=jnp.float32)
          + jnp.dot(it_ref[...], w0b_ref[...], preferred_element_type=jnp.float32))
    y0_ref[...] = y0

    @pl.when(t == 0)
    def _init():
        acc_ref[...] = jnp.zeros_like(acc_ref)

    acc_ref[0:1, :] += jnp.sum(y0, axis=0, keepdims=True)
    acc_ref[1:2, :] += jnp.sum(y0 * y0, axis=0, keepdims=True)

    @pl.when(t == n_tiles - 1)
    def _fin():
        st_ref[...] = acc_ref[...]


def _bn_coeffs(st_ref, g_ref, b_ref, inv_count):
    mean = st_ref[0:1, :] * inv_count
    var = st_ref[1:2, :] * inv_count - mean * mean
    scale = g_ref[...] * jax.lax.rsqrt(var + _BN_EPS)
    shift = b_ref[...] - mean * scale
    return scale, shift


def _stage_b_body(y0_ref, st0_ref, g0_ref, b0_ref, w1t_ref, y1_ref, st_ref,
                  acc_ref, *, n_tiles, inv_count):
    t = pl.program_id(0)
    scale, shift = _bn_coeffs(st0_ref, g0_ref, b0_ref, inv_count)
    h = jnp.maximum(y0_ref[...] * scale + shift, 0.0)
    y1 = jnp.dot(h, w1t_ref[...], preferred_element_type=jnp.float32)
    y1_ref[...] = y1

    @pl.when(t == 0)
    def _init():
        acc_ref[...] = jnp.zeros_like(acc_ref)

    acc_ref[0:1, :] += jnp.sum(y1, axis=0, keepdims=True)
    acc_ref[1:2, :] += jnp.sum(y1 * y1, axis=0, keepdims=True)

    @pl.when(t == n_tiles - 1)
    def _fin():
        st_ref[...] = acc_ref[...]


def _stage_c_body(y1_ref, st1_ref, g1_ref, b1_ref, out_ref, *, inv_count):
    scale, shift = _bn_coeffs(st1_ref, g1_ref, b1_ref, inv_count)
    out_ref[...] = jnp.maximum(y1_ref[...] * scale + shift, 0.0)


def _make_sc_gather(total_rows, d2, nc, ns):
    nw = nc * ns
    rows_per_w = total_rows // nw
    ch = 64
    n_chunks = rows_per_w // ch
    mesh = plsc.VectorSubcoreMesh(core_axis_name="c", subcore_axis_name="s")

    @functools.partial(
        pl.kernel,
        mesh=mesh,
        out_type=jax.ShapeDtypeStruct((total_rows, d2), jnp.float32),
        scratch_types=[
            pltpu.VMEM((3 * ch,), jnp.int32),
            pltpu.VMEM((3 * ch, 16), jnp.float32),
            pltpu.VMEM((3 * ch, d2), jnp.float32),
            pltpu.VMEM((ch, d2), jnp.float32),
            pltpu.SemaphoreType.DMA,
        ],
    )
    def gather_combine(p2_hbm, idx_hbm, wexp_hbm, out_hbm,
                       idx_v, w_v, rows_v, out_v, sem):
        wid = lax.axis_index("s") * nc + lax.axis_index("c")
        base = wid * rows_per_w

        def chunk(ci, carry):
            cbase = base + ci * ch
            pltpu.sync_copy(idx_hbm.at[pl.ds(3 * cbase, 3 * ch)], idx_v)
            pltpu.sync_copy(wexp_hbm.at[pl.ds(3 * cbase, 3 * ch)], w_v)
            pltpu.async_copy(p2_hbm.at[idx_v], rows_v, sem).wait()

            def row(r, carry2):
                for c in range(d2 // 16):
                    acc = None
                    for k in range(3):
                        wv = w_v[3 * r + k, :]
                        rv = rows_v[3 * r + k, pl.ds(16 * c, 16)]
                        acc = wv * rv if acc is None else acc + wv * rv
                    out_v[r, pl.ds(16 * c, 16)] = acc
                return carry2

            lax.fori_loop(0, ch, row, 0)
            pltpu.sync_copy(out_v, out_hbm.at[pl.ds(cbase, ch)])
            return carry

        lax.fori_loop(0, n_chunks, chunk, 0)

    return gather_combine


@jax.jit
def kernel(xyz1, xyz2, points1, points2, W0, gamma0, beta0, W1, gamma1, beta1):
    B, N, _ = xyz1.shape
    S = xyz2.shape[1]
    D1 = points1.shape[2]
    D2 = points2.shape[2]
    TILE_N = 512
    n_tiles_a = N // TILE_N

    xyz2t = -2.0 * jnp.transpose(xyz2, (0, 2, 1))
    w0at = W0[:, :D1].T
    w0bt = W0[:, D1:].T
    w1t = W1.T

    idx3, wexp = pl.pallas_call(
        functools.partial(_stage_a_body, s_total=S),
        grid=(B, n_tiles_a),
        in_specs=[
            pl.BlockSpec((1, TILE_N, 3), lambda b, t: (b, t, 0)),
            pl.BlockSpec((1, 3, S), lambda b, t: (b, 0, 0)),
        ],
        out_specs=[
            pl.BlockSpec((TILE_N, 3), lambda b, t, n=n_tiles_a: (b * n + t, 0)),
            pl.BlockSpec((TILE_N, 48), lambda b, t, n=n_tiles_a: (b * n + t, 0)),
        ],
        out_shape=[
            jax.ShapeDtypeStruct((B * N, 3), jnp.int32),
            jax.ShapeDtypeStruct((B * N, 48), jnp.float32),
        ],
    )(xyz1, xyz2t)

    p2f = points2.reshape(B * S, D2)
    idx_flat = idx3.reshape(B * N * 3)
    wexp_flat = wexp.reshape(B * N * 3, 16)

    sc_info = plsc.get_sparse_core_info()
    interp = _make_sc_gather(B * N, D2, sc_info.num_cores,
                             sc_info.num_subcores)(p2f, idx_flat, wexp_flat)

    inv_count = 1.0 / float(B * N)
    g0r = gamma0.reshape(1, 128)
    b0r = beta0.reshape(1, 128)
    g1r = gamma1.reshape(1, 128)
    b1r = beta1.reshape(1, 128)

    p1f = points1.reshape(B * N, D1)
    TILE_R = min(2048, B * N)
    n_tiles_b = (B * N) // TILE_R

    y0, stats0 = pl.pallas_call(
        functools.partial(_stage_a2_body, n_tiles=n_tiles_b),
        grid=(n_tiles_b,),
        in_specs=[
            pl.BlockSpec((TILE_R, D1), lambda t: (t, 0)),
            pl.BlockSpec((TILE_R, D2), lambda t: (t, 0)),
            pl.BlockSpec((D1, 128), lambda t: (0, 0)),
            pl.BlockSpec((D2, 128), lambda t: (0, 0)),
        ],
        out_specs=[
            pl.BlockSpec((TILE_R, 128), lambda t: (t, 0)),
            pl.BlockSpec((8, 128), lambda t: (0, 0)),
        ],
        out_shape=[
            jax.ShapeDtypeStruct((B * N, 128), jnp.float32),
            jax.ShapeDtypeStruct((8, 128), jnp.float32),
        ],
        scratch_shapes=[pltpu.VMEM((8, 128), jnp.float32)],
    )(p1f, interp, w0at, w0bt)

    y1, stats1 = pl.pallas_call(
        functools.partial(_stage_b_body, n_tiles=n_tiles_b,
                          inv_count=inv_count),
        grid=(n_tiles_b,),
        in_specs=[
            pl.BlockSpec((TILE_R, 128), lambda t: (t, 0)),
            pl.BlockSpec((8, 128), lambda t: (0, 0)),
            pl.BlockSpec((1, 128), lambda t: (0, 0)),
            pl.BlockSpec((1, 128), lambda t: (0, 0)),
            pl.BlockSpec((128, 128), lambda t: (0, 0)),
        ],
        out_specs=[
            pl.BlockSpec((TILE_R, 128), lambda t: (t, 0)),
            pl.BlockSpec((8, 128), lambda t: (0, 0)),
        ],
        out_shape=[
            jax.ShapeDtypeStruct((B * N, 128), jnp.float32),
            jax.ShapeDtypeStruct((8, 128), jnp.float32),
        ],
        scratch_shapes=[pltpu.VMEM((8, 128), jnp.float32)],
    )(y0, stats0, g0r, b0r, w1t)

    out = pl.pallas_call(
        functools.partial(_stage_c_body, inv_count=inv_count),
        grid=(n_tiles_b,),
        in_specs=[
            pl.BlockSpec((TILE_R, 128), lambda t: (t, 0)),
            pl.BlockSpec((8, 128), lambda t: (0, 0)),
            pl.BlockSpec((1, 128), lambda t: (0, 0)),
            pl.BlockSpec((1, 128), lambda t: (0, 0)),
        ],
        out_specs=pl.BlockSpec((TILE_R, 128), lambda t: (t, 0)),
        out_shape=jax.ShapeDtypeStruct((B * N, 128), jnp.float32),
    )(y1, stats1, g1r, b1r)

    return out.reshape(B, N, 128)


# speedup vs baseline: 1.0040x; 1.0040x over previous
import functools

import jax
import jax.numpy as jnp
from jax import lax
from jax.experimental import pallas as pl
from jax.experimental.pallas import tpu as pltpu
from jax.experimental.pallas import tpu_sc as plsc

_F32_EPS = float(jnp.finfo(jnp.float32).eps)
_BN_EPS = 1e-5


def _stage_a_body(x_ref, yt_ref, idx_ref, wexp_ref, *, s_total):
    b = pl.program_id(0)

    x = x_ref[0]
    yt = yt_ref[0]
    s = yt.shape[1]

    xx = jnp.sum(x * x, axis=1, keepdims=True)
    yy = 0.25 * jnp.sum(yt * yt, axis=0, keepdims=True)
    xy = jnp.dot(x, yt, preferred_element_type=jnp.float32)
    d = jnp.maximum((xx + yy) + xy, 0.0)

    colf = jax.lax.broadcasted_iota(jnp.int32, d.shape, 1).astype(jnp.float32)
    big = jnp.float32(s)
    ms, idxs = [], []
    for k in range(3):
        m = jnp.min(d, axis=1, keepdims=True)
        i = jnp.min(jnp.where(d == m, colf, big), axis=1, keepdims=True)
        ms.append(m)
        idxs.append(i)
        if k < 2:
            d = jnp.where(colf == i, jnp.inf, d)

    r0 = 1.0 / (jnp.sqrt(ms[0]) + _F32_EPS)
    r1 = 1.0 / (jnp.sqrt(ms[1]) + _F32_EPS)
    r2 = 1.0 / (jnp.sqrt(ms[2]) + _F32_EPS)
    norm = r0 + r1 + r2
    w0 = r0 / norm
    w1 = r1 / norm
    w2 = r2 / norm

    off = (b * s_total).astype(jnp.int32)
    ii = jnp.concatenate([idxs[0], idxs[1], idxs[2]],
                         axis=1).astype(jnp.int32) + off
    idx_ref[...] = ii

    ones16 = jnp.ones((1, 16), jnp.float32)
    wexp_ref[...] = jnp.concatenate(
        [w0 * ones16, w1 * ones16, w2 * ones16], axis=1)


def _stage_a2_body(p1_ref, it_ref, w0a_ref, w0b_ref, y0_ref, st_ref, acc_ref,
                   *, n_tiles):
    t = pl.program_id(0)
    y0 = (jnp.dot(p1_ref[...], w0a_ref[...], preferred_element_type=jnp.float32)
          + jnp.dot(it_ref[...], w0b_ref[...], preferred_element_type=jnp.float32))
    y0_ref[...] = y0

    @pl.when(t == 0)
    def _init():
        acc_ref[...] = jnp.zeros_like(acc_ref)

    acc_ref[0:1, :] += jnp.sum(y0, axis=0, keepdims=True)
    acc_ref[1:2, :] += jnp.sum(y0 * y0, axis=0, keepdims=True)

    @pl.when(t == n_tiles - 1)
    def _fin():
        st_ref[...] = acc_ref[...]


def _bn_coeffs(st_ref, g_ref, b_ref, inv_count):
    mean = st_ref[0:1, :] * inv_count
    var = st_ref[1:2, :] * inv_count - mean * mean
    scale = g_ref[...] * jax.lax.rsqrt(var + _BN_EPS)
    shift = b_ref[...] - mean * scale
    return scale, shift


def _stage_b_body(y0_ref, st0_ref, g0_ref, b0_ref, w1t_ref, y1_ref, st_ref,
                  acc_ref, *, n_tiles, inv_count):
    t = pl.program_id(0)
    scale, shift = _bn_coeffs(st0_ref, g0_ref, b0_ref, inv_count)
    h = jnp.maximum(y0_ref[...] * scale + shift, 0.0)
    y1 = jnp.dot(h, w1t_ref[...], preferred_element_type=jnp.float32)
    y1_ref[...] = y1

    @pl.when(t == 0)
    def _init():
        acc_ref[...] = jnp.zeros_like(acc_ref)

    acc_ref[0:1, :] += jnp.sum(y1, axis=0, keepdims=True)
    acc_ref[1:2, :] += jnp.sum(y1 * y1, axis=0, keepdims=True)

    @pl.when(t == n_tiles - 1)
    def _fin():
        st_ref[...] = acc_ref[...]


def _stage_c_body(y1_ref, st1_ref, g1_ref, b1_ref, out_ref, *, inv_count):
    scale, shift = _bn_coeffs(st1_ref, g1_ref, b1_ref, inv_count)
    out_ref[...] = jnp.maximum(y1_ref[...] * scale + shift, 0.0)


def _make_sc_gather(total_rows, d2, nc, ns):
    nw = nc * ns
    rows_per_w = total_rows // nw
    ch = 64
    n_chunks = rows_per_w // ch
    mesh = plsc.VectorSubcoreMesh(core_axis_name="c", subcore_axis_name="s")

    @functools.partial(
        pl.kernel,
        mesh=mesh,
        out_type=jax.ShapeDtypeStruct((total_rows, d2), jnp.float32),
        scratch_types=[
            pltpu.VMEM((3 * ch,), jnp.int32),
            pltpu.VMEM((3 * ch, 16), jnp.float32),
            pltpu.VMEM((3 * ch, d2), jnp.float32),
            pltpu.VMEM((ch, d2), jnp.float32),
            pltpu.SemaphoreType.DMA,
        ],
    )
    def gather_combine(p2_hbm, idx_hbm, wexp_hbm, out_hbm,
                       idx_v, w_v, rows_v, out_v, sem):
        wid = lax.axis_index("s") * nc + lax.axis_index("c")
        base = wid * rows_per_w

        def chunk(ci, carry):
            cbase = base + ci * ch
            pltpu.sync_copy(idx_hbm.at[pl.ds(3 * cbase, 3 * ch)], idx_v)
            pltpu.sync_copy(wexp_hbm.at[pl.ds(3 * cbase, 3 * ch)], w_v)
            pltpu.async_copy(p2_hbm.at[idx_v], rows_v, sem).wait()

            for r in range(ch):
                wv0 = w_v[3 * r, :]
                wv1 = w_v[3 * r + 1, :]
                wv2 = w_v[3 * r + 2, :]
                for c in range(d2 // 16):
                    sl = pl.ds(16 * c, 16)
                    out_v[r, sl] = (wv0 * rows_v[3 * r, sl]
                                    + wv1 * rows_v[3 * r + 1, sl]
                                    + wv2 * rows_v[3 * r + 2, sl])
            pltpu.sync_copy(out_v, out_hbm.at[pl.ds(cbase, ch)])
            return carry

        lax.fori_loop(0, n_chunks, chunk, 0)

    return gather_combine


@jax.jit
def kernel(xyz1, xyz2, points1, points2, W0, gamma0, beta0, W1, gamma1, beta1):
    B, N, _ = xyz1.shape
    S = xyz2.shape[1]
    D1 = points1.shape[2]
    D2 = points2.shape[2]
    TILE_N = 512
    n_tiles_a = N // TILE_N

    xyz2t = -2.0 * jnp.transpose(xyz2, (0, 2, 1))
    w0at = W0[:, :D1].T
    w0bt = W0[:, D1:].T
    w1t = W1.T

    idx3, wexp = pl.pallas_call(
        functools.partial(_stage_a_body, s_total=S),
        grid=(B, n_tiles_a),
        in_specs=[
            pl.BlockSpec((1, TILE_N, 3), lambda b, t: (b, t, 0)),
            pl.BlockSpec((1, 3, S), lambda b, t: (b, 0, 0)),
        ],
        out_specs=[
            pl.BlockSpec((TILE_N, 3), lambda b, t, n=n_tiles_a: (b * n + t, 0)),
            pl.BlockSpec((TILE_N, 48), lambda b, t, n=n_tiles_a: (b * n + t, 0)),
        ],
        out_shape=[
            jax.ShapeDtypeStruct((B * N, 3), jnp.int32),
            jax.ShapeDtypeStruct((B * N, 48), jnp.float32),
        ],
    )(xyz1, xyz2t)

    p2f = points2.reshape(B * S, D2)
    idx_flat = idx3.reshape(B * N * 3)
    wexp_flat = wexp.reshape(B * N * 3, 16)

    sc_info = plsc.get_sparse_core_info()
    interp = _make_sc_gather(B * N, D2, sc_info.num_cores,
                             sc_info.num_subcores)(p2f, idx_flat, wexp_flat)

    inv_count = 1.0 / float(B * N)
    g0r = gamma0.reshape(1, 128)
    b0r = beta0.reshape(1, 128)
    g1r = gamma1.reshape(1, 128)
    b1r = beta1.reshape(1, 128)

    p1f = points1.reshape(B * N, D1)
    TILE_R = min(2048, B * N)
    n_tiles_b = (B * N) // TILE_R

    y0, stats0 = pl.pallas_call(
        functools.partial(_stage_a2_body, n_tiles=n_tiles_b),
        grid=(n_tiles_b,),
        in_specs=[
            pl.BlockSpec((TILE_R, D1), lambda t: (t, 0)),
            pl.BlockSpec((TILE_R, D2), lambda t: (t, 0)),
            pl.BlockSpec((D1, 128), lambda t: (0, 0)),
            pl.BlockSpec((D2, 128), lambda t: (0, 0)),
        ],
        out_specs=[
            pl.BlockSpec((TILE_R, 128), lambda t: (t, 0)),
            pl.BlockSpec((8, 128), lambda t: (0, 0)),
        ],
        out_shape=[
            jax.ShapeDtypeStruct((B * N, 128), jnp.float32),
            jax.ShapeDtypeStruct((8, 128), jnp.float32),
        ],
        scratch_shapes=[pltpu.VMEM((8, 128), jnp.float32)],
    )(p1f, interp, w0at, w0bt)

    y1, stats1 = pl.pallas_call(
        functools.partial(_stage_b_body, n_tiles=n_tiles_b,
                          inv_count=inv_count),
        grid=(n_tiles_b,),
        in_specs=[
            pl.BlockSpec((TILE_R, 128), lambda t: (t, 0)),
            pl.BlockSpec((8, 128), lambda t: (0, 0)),
            pl.BlockSpec((1, 128), lambda t: (0, 0)),
            pl.BlockSpec((1, 128), lambda t: (0, 0)),
            pl.BlockSpec((128, 128), lambda t: (0, 0)),
        ],
        out_specs=[
            pl.BlockSpec((TILE_R, 128), lambda t: (t, 0)),
            pl.BlockSpec((8, 128), lambda t: (0, 0)),
        ],
        out_shape=[
            jax.ShapeDtypeStruct((B * N, 128), jnp.float32),
            jax.ShapeDtypeStruct((8, 128), jnp.float32),
        ],
        scratch_shapes=[pltpu.VMEM((8, 128), jnp.float32)],
    )(y0, stats0, g0r, b0r, w1t)

    out = pl.pallas_call(
        functools.partial(_stage_c_body, inv_count=inv_count),
        grid=(n_tiles_b,),
        in_specs=[
            pl.BlockSpec((TILE_R, 128), lambda t: (t, 0)),
            pl.BlockSpec((8, 128), lambda t: (0, 0)),
            pl.BlockSpec((1, 128), lambda t: (0, 0)),
            pl.BlockSpec((1, 128), lambda t: (0, 0)),
        ],
        out_specs=pl.BlockSpec((TILE_R, 128), lambda t: (t, 0)),
        out_shape=jax.ShapeDtypeStruct((B * N, 128), jnp.float32),
    )(y1, stats1, g1r, b1r)

    return out.reshape(B, N, 128)


# speedup vs baseline: 1.1549x; 1.1503x over previous
import functools

import jax
import jax.numpy as jnp
from jax import lax
from jax.experimental import pallas as pl
from jax.experimental.pallas import tpu as pltpu
from jax.experimental.pallas import tpu_sc as plsc

_F32_EPS = float(jnp.finfo(jnp.float32).eps)
_BN_EPS = 1e-5


def _stage_a_body(x_ref, yt_ref, idx_ref, w3_ref, *, s_total):
    b = pl.program_id(0)

    x = x_ref[0]
    yt = yt_ref[0]
    s = yt.shape[1]

    xx = jnp.sum(x * x, axis=1, keepdims=True)
    yy = 0.25 * jnp.sum(yt * yt, axis=0, keepdims=True)
    xy = jnp.dot(x, yt, preferred_element_type=jnp.float32)
    d = jnp.maximum((xx + yy) + xy, 0.0)

    colf = jax.lax.broadcasted_iota(jnp.int32, d.shape, 1).astype(jnp.float32)
    big = jnp.float32(s)
    ms, idxs = [], []
    for k in range(3):
        m = jnp.min(d, axis=1, keepdims=True)
        i = jnp.min(jnp.where(d == m, colf, big), axis=1, keepdims=True)
        ms.append(m)
        idxs.append(i)
        if k < 2:
            d = jnp.where(colf == i, jnp.inf, d)

    r0 = 1.0 / (jnp.sqrt(ms[0]) + _F32_EPS)
    r1 = 1.0 / (jnp.sqrt(ms[1]) + _F32_EPS)
    r2 = 1.0 / (jnp.sqrt(ms[2]) + _F32_EPS)
    norm = r0 + r1 + r2

    off = (b * s_total).astype(jnp.int32)
    ii = jnp.concatenate([idxs[0], idxs[1], idxs[2]],
                         axis=1).astype(jnp.int32) + off
    idx_ref[...] = ii
    inv_norm = 1.0 / norm
    ones16 = jnp.ones((1, 16), jnp.float32)
    w3_ref[...] = jnp.concatenate(
        [(r0 * inv_norm) * ones16, (r1 * inv_norm) * ones16,
         (r2 * inv_norm) * ones16], axis=1)


def _stage_a2_body(p1_ref, it_ref, w0a_ref, w0b_ref, y0_ref, st_ref, acc_ref,
                   *, n_tiles):
    t = pl.program_id(0)
    y0 = (jnp.dot(p1_ref[...], w0a_ref[...], preferred_element_type=jnp.float32)
          + jnp.dot(it_ref[...], w0b_ref[...], preferred_element_type=jnp.float32))
    y0_ref[...] = y0

    @pl.when(t == 0)
    def _init():
        acc_ref[...] = jnp.zeros_like(acc_ref)

    acc_ref[0:1, :] += jnp.sum(y0, axis=0, keepdims=True)
    acc_ref[1:2, :] += jnp.sum(y0 * y0, axis=0, keepdims=True)

    @pl.when(t == n_tiles - 1)
    def _fin():
        st_ref[...] = acc_ref[...]


def _bn_coeffs(st_ref, g_ref, b_ref, inv_count):
    mean = st_ref[0:1, :] * inv_count
    var = st_ref[1:2, :] * inv_count - mean * mean
    scale = g_ref[...] * jax.lax.rsqrt(var + _BN_EPS)
    shift = b_ref[...] - mean * scale
    return scale, shift


def _stage_b_body(y0_ref, st0_ref, g0_ref, b0_ref, w1t_ref, y1_ref, st_ref,
                  acc_ref, *, n_tiles, inv_count):
    t = pl.program_id(0)
    scale, shift = _bn_coeffs(st0_ref, g0_ref, b0_ref, inv_count)
    h = jnp.maximum(y0_ref[...] * scale + shift, 0.0)
    y1 = jnp.dot(h, w1t_ref[...], preferred_element_type=jnp.float32)
    y1_ref[...] = y1

    @pl.when(t == 0)
    def _init():
        acc_ref[...] = jnp.zeros_like(acc_ref)

    acc_ref[0:1, :] += jnp.sum(y1, axis=0, keepdims=True)
    acc_ref[1:2, :] += jnp.sum(y1 * y1, axis=0, keepdims=True)

    @pl.when(t == n_tiles - 1)
    def _fin():
        st_ref[...] = acc_ref[...]


def _stage_c_body(y1_ref, st1_ref, g1_ref, b1_ref, out_ref, *, inv_count):
    scale, shift = _bn_coeffs(st1_ref, g1_ref, b1_ref, inv_count)
    out_ref[...] = jnp.maximum(y1_ref[...] * scale + shift, 0.0)


def _make_sc_gather(total_rows, d2, nc, ns):
    nw = nc * ns
    rows_per_w = total_rows // nw
    ch = 32
    n_chunks = rows_per_w // ch
    n_pairs = n_chunks // 2
    mesh = plsc.VectorSubcoreMesh(core_axis_name="c", subcore_axis_name="s")

    @functools.partial(
        pl.kernel,
        mesh=mesh,
        out_type=jax.ShapeDtypeStruct((total_rows, d2), jnp.float32),
        scratch_types=[
            pltpu.VMEM((3 * rows_per_w,), jnp.int32),
            pltpu.VMEM((2, 3 * ch, 16), jnp.float32),
            pltpu.VMEM((2, 3 * ch, d2), jnp.float32),
            pltpu.VMEM((2, ch, d2), jnp.float32),
            pltpu.SemaphoreType.DMA,
            pltpu.SemaphoreType.DMA,
            pltpu.SemaphoreType.DMA,
            pltpu.SemaphoreType.DMA,
            pltpu.SemaphoreType.DMA,
            pltpu.SemaphoreType.DMA,
        ],
    )
    def gather_combine(p2_hbm, idx_hbm, wexp_hbm, out_hbm,
                       idx_v, w_v, rows_v, out_v, sg0, sg1, sw0, sw1, so0, so1):
        wid = lax.axis_index("s") * nc + lax.axis_index("c")
        base = wid * rows_per_w
        pltpu.sync_copy(idx_hbm.at[pl.ds(3 * base, 3 * rows_per_w)], idx_v)

        def start_gather(ci, slot, sem, wsem):
            pltpu.async_copy(
                p2_hbm.at[idx_v.at[pl.ds(ci * 3 * ch, 3 * ch)]],
                rows_v.at[slot], sem)
            pltpu.async_copy(
                wexp_hbm.at[pl.ds(3 * (base + ci * ch), 3 * ch)],
                w_v.at[slot], wsem)

        def drain_gather(slot, sem, wsem):
            pltpu.make_async_copy(p2_hbm.at[pl.ds(0, 3 * ch)],
                                  rows_v.at[slot], sem).wait()
            pltpu.make_async_copy(wexp_hbm.at[pl.ds(0, 3 * ch)],
                                  w_v.at[slot], wsem).wait()

        def drain_out(slot, sem):
            pltpu.make_async_copy(out_hbm.at[pl.ds(0, ch)],
                                  out_v.at[slot], sem).wait()

        def compute(ci, slot, sem):
            for r in range(ch):
                w0 = w_v[slot, 3 * r, :]
                w1 = w_v[slot, 3 * r + 1, :]
                w2 = w_v[slot, 3 * r + 2, :]
                for c in range(d2 // 16):
                    sl = pl.ds(16 * c, 16)
                    out_v[slot, r, sl] = (w0 * rows_v[slot, 3 * r, sl]
                                          + w1 * rows_v[slot, 3 * r + 1, sl]
                                          + w2 * rows_v[slot, 3 * r + 2, sl])
            pltpu.async_copy(out_v.at[slot],
                             out_hbm.at[pl.ds(base + ci * ch, ch)], sem)

        start_gather(0, 0, sg0, sw0)

        def pair(p, carry):
            c0 = 2 * p
            start_gather(c0 + 1, 1, sg1, sw1)
            drain_gather(0, sg0, sw0)

            @pl.when(p > 0)
            def _d0():
                drain_out(0, so0)

            compute(c0, 0, so0)
            nxt = jnp.minimum(c0 + 2, n_chunks - 1)
            start_gather(nxt, 0, sg0, sw0)
            drain_gather(1, sg1, sw1)

            @pl.when(p > 0)
            def _d1():
                drain_out(1, so1)

            compute(c0 + 1, 1, so1)
            return carry

        lax.fori_loop(0, n_pairs, pair, 0)
        drain_gather(0, sg0, sw0)
        drain_out(0, so0)
        drain_out(1, so1)

    return gather_combine


@jax.jit
def kernel(xyz1, xyz2, points1, points2, W0, gamma0, beta0, W1, gamma1, beta1):
    B, N, _ = xyz1.shape
    S = xyz2.shape[1]
    D1 = points1.shape[2]
    D2 = points2.shape[2]
    TILE_N = 512
    n_tiles_a = N // TILE_N

    xyz2t = -2.0 * jnp.transpose(xyz2, (0, 2, 1))
    w0at = W0[:, :D1].T
    w0bt = W0[:, D1:].T
    w1t = W1.T

    idx3, w3 = pl.pallas_call(
        functools.partial(_stage_a_body, s_total=S),
        grid=(B, n_tiles_a),
        in_specs=[
            pl.BlockSpec((1, TILE_N, 3), lambda b, t: (b, t, 0)),
            pl.BlockSpec((1, 3, S), lambda b, t: (b, 0, 0)),
        ],
        out_specs=[
            pl.BlockSpec((TILE_N, 3), lambda b, t, n=n_tiles_a: (b * n + t, 0)),
            pl.BlockSpec((TILE_N, 48), lambda b, t, n=n_tiles_a: (b * n + t, 0)),
        ],
        out_shape=[
            jax.ShapeDtypeStruct((B * N, 3), jnp.int32),
            jax.ShapeDtypeStruct((B * N, 48), jnp.float32),
        ],
    )(xyz1, xyz2t)

    p2f = points2.reshape(B * S, D2)
    idx_flat = idx3.reshape(B * N * 3)
    w_flat = w3.reshape(B * N * 3, 16)

    sc_info = plsc.get_sparse_core_info()
    interp = _make_sc_gather(B * N, D2, sc_info.num_cores,
                             sc_info.num_subcores)(p2f, idx_flat, w_flat)

    inv_count = 1.0 / float(B * N)
    g0r = gamma0.reshape(1, 128)
    b0r = beta0.reshape(1, 128)
    g1r = gamma1.reshape(1, 128)
    b1r = beta1.reshape(1, 128)

    p1f = points1.reshape(B * N, D1)
    TILE_R = min(2048, B * N)
    n_tiles_b = (B * N) // TILE_R

    y0, stats0 = pl.pallas_call(
        functools.partial(_stage_a2_body, n_tiles=n_tiles_b),
        grid=(n_tiles_b,),
        in_specs=[
            pl.BlockSpec((TILE_R, D1), lambda t: (t, 0)),
            pl.BlockSpec((TILE_R, D2), lambda t: (t, 0)),
            pl.BlockSpec((D1, 128), lambda t: (0, 0)),
            pl.BlockSpec((D2, 128), lambda t: (0, 0)),
        ],
        out_specs=[
            pl.BlockSpec((TILE_R, 128), lambda t: (t, 0)),
            pl.BlockSpec((8, 128), lambda t: (0, 0)),
        ],
        out_shape=[
            jax.ShapeDtypeStruct((B * N, 128), jnp.float32),
            jax.ShapeDtypeStruct((8, 128), jnp.float32),
        ],
        scratch_shapes=[pltpu.VMEM((8, 128), jnp.float32)],
    )(p1f, interp, w0at, w0bt)

    y1, stats1 = pl.pallas_call(
        functools.partial(_stage_b_body, n_tiles=n_tiles_b,
                          inv_count=inv_count),
        grid=(n_tiles_b,),
        in_specs=[
            pl.BlockSpec((TILE_R, 128), lambda t: (t, 0)),
            pl.BlockSpec((8, 128), lambda t: (0, 0)),
            pl.BlockSpec((1, 128), lambda t: (0, 0)),
            pl.BlockSpec((1, 128), lambda t: (0, 0)),
            pl.BlockSpec((128, 128), lambda t: (0, 0)),
        ],
        out_specs=[
            pl.BlockSpec((TILE_R, 128), lambda t: (t, 0)),
            pl.BlockSpec((8, 128), lambda t: (0, 0)),
        ],
        out_shape=[
            jax.ShapeDtypeStruct((B * N, 128), jnp.float32),
            jax.ShapeDtypeStruct((8, 128), jnp.float32),
        ],
        scratch_shapes=[pltpu.VMEM((8, 128), jnp.float32)],
    )(y0, stats0, g0r, b0r, w1t)

    out = pl.pallas_call(
        functools.partial(_stage_c_body, inv_count=inv_count),
        grid=(n_tiles_b,),
        in_specs=[
            pl.BlockSpec((TILE_R, 128), lambda t: (t, 0)),
            pl.BlockSpec((8, 128), lambda t: (0, 0)),
            pl.BlockSpec((1, 128), lambda t: (0, 0)),
            pl.BlockSpec((1, 128), lambda t: (0, 0)),
        ],
        out_specs=pl.BlockSpec((TILE_R, 128), lambda t: (t, 0)),
        out_shape=jax.ShapeDtypeStruct((B * N, 128), jnp.float32),
    )(y1, stats1, g1r, b1r)

    return out.reshape(B, N, 128)
